# Initial kernel scaffold; baseline (speedup 1.0000x reference)
import functools

import jax
import jax.numpy as jnp
from jax.experimental import pallas as pl
from jax.experimental.pallas import tpu as pltpu

E_BLOCK = 2048


def _t1_body(ef_ref, wea_ref, bea_ref, wm1_ref, bm1_ref, ei_ref, e1_ref):
    ef = ef_ref[...]
    ei_ref[...] = jnp.dot(ef, wea_ref[...],
                          preferred_element_type=jnp.float32) + bea_ref[...]
    e1_ref[...] = jnp.maximum(
        jnp.dot(ef, wm1_ref[...], preferred_element_type=jnp.float32)
        + bm1_ref[...], 0.0)


def _t1(edge_feats, W_ea, b_ea, W_m1, b_m1):
    E, D = edge_feats.shape
    H = W_ea.shape[1]
    grid = (E // E_BLOCK,)
    return pl.pallas_call(
        _t1_body,
        grid=grid,
        in_specs=[
            pl.BlockSpec((E_BLOCK, D), lambda i: (i, 0)),
            pl.BlockSpec((D, H), lambda i: (0, 0)),
            pl.BlockSpec((1, H), lambda i: (0, 0)),
            pl.BlockSpec((D, H), lambda i: (0, 0)),
            pl.BlockSpec((1, H), lambda i: (0, 0)),
        ],
        out_specs=[
            pl.BlockSpec((E_BLOCK, H), lambda i: (i, 0)),
            pl.BlockSpec((E_BLOCK, H), lambda i: (i, 0)),
        ],
        out_shape=[
            jax.ShapeDtypeStruct((E, H), jnp.float32),
            jax.ShapeDtypeStruct((E, H), jnp.float32),
        ],
    )(edge_feats, W_ea, b_ea.reshape(1, H), W_m1, b_m1.reshape(1, H))


def _t2a_body(sr_ref, cr_ref, sc_ref, cc_ref, nf_ref, wg1_ref,
              agg_ref, dinv_ref, xw1_ref, y1_ref):
    cr = jnp.maximum(cr_ref[...], 1.0)
    cc = jnp.maximum(cc_ref[...], 1.0)
    raw_row = sr_ref[...] / cr
    raw_col = sc_ref[...] / cc
    agg_ref[...] = jax.nn.sigmoid(raw_row * raw_col)
    dinv = jax.lax.rsqrt(cc_ref[...] + 1.0)
    dinv_ref[...] = dinv
    xw1 = jnp.dot(nf_ref[...], wg1_ref[...], preferred_element_type=jnp.float32)
    xw1_ref[...] = xw1
    y1_ref[...] = xw1 * dinv


def _t2a(sum_row, cnt_row, sum_col, cnt_col, node_feats, W_g1):
    N, H = sum_row.shape
    return pl.pallas_call(
        _t2a_body,
        out_shape=[
            jax.ShapeDtypeStruct((N, H), jnp.float32),
            jax.ShapeDtypeStruct((N, 1), jnp.float32),
            jax.ShapeDtypeStruct((N, H), jnp.float32),
            jax.ShapeDtypeStruct((N, H), jnp.float32),
        ],
    )(sum_row, cnt_row, sum_col, cnt_col, node_feats, W_g1)


def _t2b_body(s1a_ref, s1b_ref, dinv_ref, xw1_ref, agg_ref, wg2_ref, bg1_ref,
              xw2_ref, y2_ref):
    dinv = dinv_ref[...]
    s1 = s1a_ref[...] + s1b_ref[...]
    out1 = s1 * dinv + xw1_ref[...] * (dinv * dinv) + bg1_ref[...]
    x1 = jnp.maximum(out1, 0.0) * agg_ref[...]
    xw2 = jnp.dot(x1, wg2_ref[...], preferred_element_type=jnp.float32)
    xw2_ref[...] = xw2
    y2_ref[...] = xw2 * dinv


def _t2b(S1a, S1b, dinv, xw1, agg, W_g2, b_g1):
    N, H = xw1.shape
    D = W_g2.shape[1]
    return pl.pallas_call(
        _t2b_body,
        out_shape=[
            jax.ShapeDtypeStruct((N, D), jnp.float32),
            jax.ShapeDtypeStruct((N, D), jnp.float32),
        ],
    )(S1a, S1b, dinv, xw1, agg, W_g2, b_g1.reshape(1, H))


def _t2c_body(s2a_ref, s2b_ref, dinv_ref, xw2_ref, wna_ref, bna_ref,
              wnra_ref, wnrb_ref, bnr_ref, bg2_ref,
              x_ref, a_ref, b_ref):
    dinv = dinv_ref[...]
    s2 = s2a_ref[...] + s2b_ref[...]
    x = jnp.maximum(s2 * dinv + xw2_ref[...] * (dinv * dinv) + bg2_ref[...],
                    0.0)
    x_ref[...] = x
    node_ind = jnp.maximum(
        jnp.dot(x, wna_ref[...], preferred_element_type=jnp.float32)
        + bna_ref[...], 0.0)
    a_ref[...] = jnp.dot(node_ind, wnra_ref[...],
                         preferred_element_type=jnp.float32) + bnr_ref[...]
    b_ref[...] = jnp.dot(node_ind, wnrb_ref[...],
                         preferred_element_type=jnp.float32)


def _t2c(S2a, S2b, dinv, xw2, W_na, b_na, W_nrA, W_nrB, b_nr, b_g2):
    N, D = xw2.shape
    H = W_na.shape[1]
    return pl.pallas_call(
        _t2c_body,
        out_shape=[
            jax.ShapeDtypeStruct((N, D), jnp.float32),
            jax.ShapeDtypeStruct((N, H), jnp.float32),
            jax.ShapeDtypeStruct((N, H), jnp.float32),
        ],
    )(S2a, S2b, dinv, xw2, W_na, b_na.reshape(1, H),
      W_nrA, W_nrB, b_nr.reshape(1, H), b_g2.reshape(1, D))


def _t3_body(e1_ref, p_ref, wm2_ref, bm2_ref, e_ref):
    g = e1_ref[...] * jax.nn.sigmoid(p_ref[...])
    e_ref[...] = jnp.maximum(
        jnp.dot(g, wm2_ref[...], preferred_element_type=jnp.float32)
        + bm2_ref[...], 0.0)


def _t3(e1, P, W_m2, b_m2):
    E, H = e1.shape
    D = W_m2.shape[1]
    grid = (E // E_BLOCK,)
    return pl.pallas_call(
        _t3_body,
        grid=grid,
        in_specs=[
            pl.BlockSpec((E_BLOCK, H), lambda i: (i, 0)),
            pl.BlockSpec((E_BLOCK, H), lambda i: (i, 0)),
            pl.BlockSpec((H, D), lambda i: (0, 0)),
            pl.BlockSpec((1, D), lambda i: (0, 0)),
        ],
        out_specs=pl.BlockSpec((E_BLOCK, D), lambda i: (i, 0)),
        out_shape=jax.ShapeDtypeStruct((E, D), jnp.float32),
    )(e1, P, W_m2, b_m2.reshape(1, D))


def _k1_scatter_stats(edge_ind, src, dst, N):
    H = edge_ind.shape[1]
    sum_row = jnp.zeros((N, H), jnp.float32).at[src].add(edge_ind)
    cnt_row = jnp.zeros((N, 1), jnp.float32).at[src, 0].add(1.0)
    sum_col = jnp.zeros((N, H), jnp.float32).at[dst].add(edge_ind)
    cnt_col = jnp.zeros((N, 1), jnp.float32).at[dst, 0].add(1.0)
    return sum_row, cnt_row, sum_col, cnt_col


def _k23_message(y, src, dst, N):
    S = jnp.zeros_like(y).at[dst].add(y[src])
    return S, jnp.zeros_like(S)


def _k4_pair_gather(A, B, src, dst):
    return A[src] + B[dst]


def kernel(node_feats, edge_feats, edge_index, W_g1, b_g1, W_g2, b_g2,
           W_ea, b_ea, W_na, b_na, W_nr, b_nr, W_m1, b_m1, W_m2, b_m2):
    src = edge_index[0]
    dst = edge_index[1]
    N, D = node_feats.shape
    H = D // 2

    edge_ind, e1 = _t1(edge_feats, W_ea, b_ea, W_m1, b_m1)

    sum_row, cnt_row, sum_col, cnt_col = _k1_scatter_stats(edge_ind, src, dst, N)
    agg, dinv, xw1, y1 = _t2a(sum_row, cnt_row, sum_col, cnt_col,
                              node_feats, W_g1)

    S1a, S1b = _k23_message(y1, src, dst, N)
    xw2, y2 = _t2b(S1a, S1b, dinv, xw1, agg, W_g2, b_g1)

    S2a, S2b = _k23_message(y2, src, dst, N)
    x, A, B = _t2c(S2a, S2b, dinv, xw2, W_na, b_na,
                   W_nr[:H], W_nr[H:], b_nr, b_g2)

    P = _k4_pair_gather(A, B, src, dst)
    e = _t3(e1, P, W_m2, b_m2)
    return (x, e)


# speedup vs baseline: 2.0401x; 2.0401x over previous
import functools

import jax
import jax.numpy as jnp
from jax.experimental import pallas as pl
from jax.experimental.pallas import tpu as pltpu

E_BLOCK = 3200


def _t1_body(ef_ref, wea_ref, bea_ref, wm1_ref, bm1_ref, ei_ref, e1_ref):
    ef = ef_ref[...]
    ei_ref[...] = jnp.dot(ef, wea_ref[...],
                          preferred_element_type=jnp.float32) + bea_ref[...]
    e1_ref[...] = jnp.maximum(
        jnp.dot(ef, wm1_ref[...], preferred_element_type=jnp.float32)
        + bm1_ref[...], 0.0)


def _t1(edge_feats, W_ea, b_ea, W_m1, b_m1):
    E, D = edge_feats.shape
    H = W_ea.shape[1]
    grid = (E // E_BLOCK,)
    return pl.pallas_call(
        _t1_body,
        grid=grid,
        in_specs=[
            pl.BlockSpec((E_BLOCK, D), lambda i: (i, 0)),
            pl.BlockSpec((D, H), lambda i: (0, 0)),
            pl.BlockSpec((1, H), lambda i: (0, 0)),
            pl.BlockSpec((D, H), lambda i: (0, 0)),
            pl.BlockSpec((1, H), lambda i: (0, 0)),
        ],
        out_specs=[
            pl.BlockSpec((E_BLOCK, H), lambda i: (i, 0)),
            pl.BlockSpec((E_BLOCK, H), lambda i: (i, 0)),
        ],
        out_shape=[
            jax.ShapeDtypeStruct((E, H), jnp.float32),
            jax.ShapeDtypeStruct((E, H), jnp.float32),
        ],
    )(edge_feats, W_ea, b_ea.reshape(1, H), W_m1, b_m1.reshape(1, H))


def _t2a_body(sr_ref, cr_ref, sc_ref, cc_ref, nf_ref, wg1_ref,
              agg_ref, dinv_ref, xw1_ref, y1_ref):
    cr = jnp.maximum(cr_ref[...], 1.0)
    cc = jnp.maximum(cc_ref[...], 1.0)
    raw_row = sr_ref[...] / cr
    raw_col = sc_ref[...] / cc
    agg_ref[...] = jax.nn.sigmoid(raw_row * raw_col)
    dinv = jax.lax.rsqrt(cc_ref[...] + 1.0)
    dinv_ref[...] = dinv
    xw1 = jnp.dot(nf_ref[...], wg1_ref[...], preferred_element_type=jnp.float32)
    xw1_ref[...] = xw1
    y1_ref[...] = xw1 * dinv


def _t2a(sum_row, cnt_row, sum_col, cnt_col, node_feats, W_g1):
    N, H = sum_row.shape
    return pl.pallas_call(
        _t2a_body,
        out_shape=[
            jax.ShapeDtypeStruct((N, H), jnp.float32),
            jax.ShapeDtypeStruct((N, 1), jnp.float32),
            jax.ShapeDtypeStruct((N, H), jnp.float32),
            jax.ShapeDtypeStruct((N, H), jnp.float32),
        ],
    )(sum_row, cnt_row, sum_col, cnt_col, node_feats, W_g1)


def _t2b_body(s1a_ref, s1b_ref, dinv_ref, xw1_ref, agg_ref, wg2_ref, bg1_ref,
              xw2_ref, y2_ref):
    dinv = dinv_ref[...]
    s1 = s1a_ref[...] + s1b_ref[...]
    out1 = s1 * dinv + xw1_ref[...] * (dinv * dinv) + bg1_ref[...]
    x1 = jnp.maximum(out1, 0.0) * agg_ref[...]
    xw2 = jnp.dot(x1, wg2_ref[...], preferred_element_type=jnp.float32)
    xw2_ref[...] = xw2
    y2_ref[...] = xw2 * dinv


def _t2b(S1a, S1b, dinv, xw1, agg, W_g2, b_g1):
    N, H = xw1.shape
    D = W_g2.shape[1]
    return pl.pallas_call(
        _t2b_body,
        out_shape=[
            jax.ShapeDtypeStruct((N, D), jnp.float32),
            jax.ShapeDtypeStruct((N, D), jnp.float32),
        ],
    )(S1a, S1b, dinv, xw1, agg, W_g2, b_g1.reshape(1, H))


def _t2c_body(s2a_ref, s2b_ref, dinv_ref, xw2_ref, wna_ref, bna_ref,
              wnra_ref, wnrb_ref, bnr_ref, bg2_ref,
              x_ref, a_ref, b_ref):
    dinv = dinv_ref[...]
    s2 = s2a_ref[...] + s2b_ref[...]
    x = jnp.maximum(s2 * dinv + xw2_ref[...] * (dinv * dinv) + bg2_ref[...],
                    0.0)
    x_ref[...] = x
    node_ind = jnp.maximum(
        jnp.dot(x, wna_ref[...], preferred_element_type=jnp.float32)
        + bna_ref[...], 0.0)
    a_ref[...] = jnp.dot(node_ind, wnra_ref[...],
                         preferred_element_type=jnp.float32) + bnr_ref[...]
    b_ref[...] = jnp.dot(node_ind, wnrb_ref[...],
                         preferred_element_type=jnp.float32)


def _t2c(S2a, S2b, dinv, xw2, W_na, b_na, W_nrA, W_nrB, b_nr, b_g2):
    N, D = xw2.shape
    H = W_na.shape[1]
    return pl.pallas_call(
        _t2c_body,
        out_shape=[
            jax.ShapeDtypeStruct((N, D), jnp.float32),
            jax.ShapeDtypeStruct((N, H), jnp.float32),
            jax.ShapeDtypeStruct((N, H), jnp.float32),
        ],
    )(S2a, S2b, dinv, xw2, W_na, b_na.reshape(1, H),
      W_nrA, W_nrB, b_nr.reshape(1, H), b_g2.reshape(1, D))


def _t3_body(e1_ref, p_ref, wm2_ref, bm2_ref, e_ref):
    g = e1_ref[...] * jax.nn.sigmoid(p_ref[...])
    e_ref[...] = jnp.maximum(
        jnp.dot(g, wm2_ref[...], preferred_element_type=jnp.float32)
        + bm2_ref[...], 0.0)


def _t3(e1, P, W_m2, b_m2):
    E, H = e1.shape
    D = W_m2.shape[1]
    grid = (E // E_BLOCK,)
    return pl.pallas_call(
        _t3_body,
        grid=grid,
        in_specs=[
            pl.BlockSpec((E_BLOCK, H), lambda i: (i, 0)),
            pl.BlockSpec((E_BLOCK, H), lambda i: (i, 0)),
            pl.BlockSpec((H, D), lambda i: (0, 0)),
            pl.BlockSpec((1, D), lambda i: (0, 0)),
        ],
        out_specs=pl.BlockSpec((E_BLOCK, D), lambda i: (i, 0)),
        out_shape=jax.ShapeDtypeStruct((E, D), jnp.float32),
    )(e1, P, W_m2, b_m2.reshape(1, D))


def _k1_scatter_stats(edge_ind, src, dst, N):
    H = edge_ind.shape[1]
    sum_row = jnp.zeros((N, H), jnp.float32).at[src].add(edge_ind)
    cnt_row = jnp.zeros((N, 1), jnp.float32).at[src, 0].add(1.0)
    sum_col = jnp.zeros((N, H), jnp.float32).at[dst].add(edge_ind)
    cnt_col = jnp.zeros((N, 1), jnp.float32).at[dst, 0].add(1.0)
    return sum_row, cnt_row, sum_col, cnt_col


def _k23_message(y, src, dst, N):
    S = jnp.zeros_like(y).at[dst].add(y[src])
    return S, jnp.zeros_like(S)


def _k4_pair_gather(A, B, src, dst):
    return A[src] + B[dst]


def kernel(node_feats, edge_feats, edge_index, W_g1, b_g1, W_g2, b_g2,
           W_ea, b_ea, W_na, b_na, W_nr, b_nr, W_m1, b_m1, W_m2, b_m2):
    src = edge_index[0]
    dst = edge_index[1]
    N, D = node_feats.shape
    H = D // 2

    edge_ind, e1 = _t1(edge_feats, W_ea, b_ea, W_m1, b_m1)

    sum_row, cnt_row, sum_col, cnt_col = _k1_scatter_stats(edge_ind, src, dst, N)
    agg, dinv, xw1, y1 = _t2a(sum_row, cnt_row, sum_col, cnt_col,
                              node_feats, W_g1)

    S1a, S1b = _k23_message(y1, src, dst, N)
    xw2, y2 = _t2b(S1a, S1b, dinv, xw1, agg, W_g2, b_g1)

    S2a, S2b = _k23_message(y2, src, dst, N)
    x, A, B = _t2c(S2a, S2b, dinv, xw2, W_na, b_na,
                   W_nr[:H], W_nr[H:], b_nr, b_g2)

    P = _k4_pair_gather(A, B, src, dst)
    e = _t3(e1, P, W_m2, b_m2)
    return (x, e)
